# Initial kernel scaffold; baseline (speedup 1.0000x reference)
import jax
import jax.numpy as jnp
from jax.experimental import pallas as pl


def kernel(node_features, node_to_graphid, W_up, b_up, W_gate, b_gate, W_func, b_func):
    raise NotImplementedError("write your pallas kernel here")


# speedup vs baseline: 2.2466x; 2.2466x over previous
import jax
import jax.numpy as jnp
from jax import lax
from jax.experimental import pallas as pl
from jax.experimental.pallas import tpu as pltpu
from jax.experimental.pallas import tpu_sc as plsc

N_NODES = 100000
D_FEAT = 128
TWO_D = 256
FINAL_DIM = 128
NUM_GRAPHS = 512

NC = 2
NS = 16
NW = NC * NS
G_PER_W = NUM_GRAPHS // NW
CHUNK = 112
GATED_ROWS = 100480
BLK = 3136
LANES = 16
NL = TWO_D // LANES
INT_MIN = -2147483648


def _gate_up_body(x_ref, w_ref, b_ref, out_ref):
    y = jnp.dot(x_ref[...], w_ref[...], preferred_element_type=jnp.float32)
    y = y + b_ref[...]
    gate = jax.nn.sigmoid(y[:, TWO_D:TWO_D + 1])
    out_ref[...] = y[:, :TWO_D] * gate


def _segment_sum_body(gated_hbm, bnd_hbm, out_hbm, bndv, rows_v, outbuf):
    c = lax.axis_index("c")
    s = lax.axis_index("s")
    w = c * NS + s
    pltpu.sync_copy(bnd_hbm.at[pl.ds(w * G_PER_W, 32)], bndv)
    lo16 = bndv[pl.ds(0, LANES)]
    hi16 = bndv[pl.ds(LANES, LANES)]
    b_list = [lo16[j] for j in range(LANES)] + [hi16[0]]

    for g in range(G_PER_W):
        a = b_list[g]
        b = b_list[g + 1]
        start0 = pl.multiple_of((a // 8) * 8, 8)
        nch = (b - start0 + CHUNK - 1) // CHUNK
        accs = tuple(jnp.zeros((LANES,), jnp.float32) for _ in range(NL))

        def chunk_body(ck, accs, a=a, b=b, start0=start0):
            st = pl.multiple_of(start0 + ck * CHUNK, 8)
            pltpu.sync_copy(gated_hbm.at[pl.ds(st, CHUNK)], rows_v)
            lo = jnp.maximum(a - st, 0)
            hi = jnp.minimum(b - st, CHUNK)

            def row_body(r, accs):
                return tuple(accs[l] + rows_v[r, pl.ds(l * LANES, LANES)]
                             for l in range(NL))

            return lax.fori_loop(lo, hi, row_body, accs)

        accs = lax.fori_loop(0, nch, chunk_body, accs)
        for l in range(NL):
            outbuf[g, pl.ds(l * LANES, LANES)] = accs[l]
    pltpu.sync_copy(outbuf, out_hbm.at[pl.ds(w * G_PER_W, G_PER_W)])


def _final_body(p_ref, w_ref, b_ref, out_ref):
    out_ref[...] = (
        jnp.dot(p_ref[...], w_ref[...], preferred_element_type=jnp.float32)
        + b_ref[...])


def kernel(node_features, node_to_graphid, W_up, b_up, W_gate, b_gate,
           W_func, b_func):
    w_cat = jnp.concatenate(
        [W_up, jnp.pad(W_gate, ((0, 0), (0, D_FEAT - 1)))], axis=1)
    b_cat = jnp.concatenate(
        [b_up, b_gate, jnp.zeros((D_FEAT - 1,), jnp.float32)])[None, :]
    ids = node_to_graphid.astype(jnp.int32)
    bnd = jnp.searchsorted(
        ids, jnp.arange(NUM_GRAPHS + 1, dtype=jnp.int32)).astype(jnp.int32)
    bnd = jnp.pad(bnd, (0, 544 - (NUM_GRAPHS + 1)),
                  constant_values=N_NODES)

    gated = pl.pallas_call(
        _gate_up_body,
        grid=(BLK * 32 // BLK,),
        in_specs=[
            pl.BlockSpec((BLK, D_FEAT), lambda i: (i, 0)),
            pl.BlockSpec((D_FEAT, TWO_D + D_FEAT), lambda i: (0, 0)),
            pl.BlockSpec((1, TWO_D + D_FEAT), lambda i: (0, 0)),
        ],
        out_specs=pl.BlockSpec((BLK, TWO_D), lambda i: (i, 0)),
        out_shape=jax.ShapeDtypeStruct((GATED_ROWS, TWO_D), jnp.float32),
    )(node_features, w_cat, b_cat)

    seg = pl.kernel(
        _segment_sum_body,
        out_type=jax.ShapeDtypeStruct((NUM_GRAPHS, TWO_D), jnp.float32),
        mesh=plsc.VectorSubcoreMesh(core_axis_name="c", subcore_axis_name="s"),
        scratch_types=[
            pltpu.VMEM((32,), jnp.int32),
            pltpu.VMEM((CHUNK, TWO_D), jnp.float32),
            pltpu.VMEM((G_PER_W, TWO_D), jnp.float32),
        ],
    )(gated, bnd)

    out = pl.pallas_call(
        _final_body,
        grid=(1,),
        in_specs=[
            pl.BlockSpec((NUM_GRAPHS, TWO_D), lambda i: (0, 0)),
            pl.BlockSpec((TWO_D, FINAL_DIM), lambda i: (0, 0)),
            pl.BlockSpec((1, FINAL_DIM), lambda i: (0, 0)),
        ],
        out_specs=pl.BlockSpec((NUM_GRAPHS, FINAL_DIM), lambda i: (0, 0)),
        out_shape=jax.ShapeDtypeStruct((NUM_GRAPHS, FINAL_DIM), jnp.float32),
    )(seg, W_func, b_func[None, :])
    return out
